# baseline (speedup 1.0000x reference)
import jax
import jax.numpy as jnp
from jax import lax
from jax.experimental import pallas as pl
from jax.experimental.pallas import tpu as pltpu
from jax.experimental.pallas import tpu_sc as plsc

IN_C = 256
HID = 256
OUT_C = 128
N0, N1, N2 = 10000, 5000, 2500
E0, E1 = 160000, 80000
NCLUST = 64

NC = 2
NS = 16
NW = NC * NS
CHUNK = 128

N1P = 5008
N2P = 2512
FG = 16


def _sc_agg(nrows_pad, nchunks):
    NG = CHUNK // 16
    KPG = FG

    def body(tablef, src_hbm, dst_hbm, acc_out, cnt_out,
             sidx, fidx, didx, rows16, acc_v, cnt_v, sem):
        c = lax.axis_index("c")
        t = lax.axis_index("s")

        zero16 = jnp.zeros((16,), jnp.float32)
        ones16 = jnp.ones((16,), jnp.float32)
        lane16 = lax.iota(jnp.int32, 16)
        t16 = jnp.broadcast_to(t, (16,)).astype(jnp.int32)

        def za(i, carry):
            for u in range(4):
                acc_v[i * 4 + u, pl.ds(0, 16)] = zero16
            return carry

        lax.fori_loop(0, nrows_pad // 4, za, 0)

        def zc(i, carry):
            cnt_v[pl.ds(i * 16, 16)] = zero16
            return carry

        lax.fori_loop(0, nrows_pad // 16, zc, 0)

        base = c * nchunks

        def chunk_body(i, carry):
            pltpu.sync_copy(src_hbm.at[base + i], sidx)
            pltpu.sync_copy(dst_hbm.at[base + i], didx)
            for g in range(NG):
                gs = pl.ds(g * 16, 16)
                fidx[gs] = sidx[gs] * 16 + t16
            pltpu.async_copy(tablef.at[fidx], rows16, sem).wait()
            for g in range(NG):
                dj16 = didx[pl.ds(g * 16, 16)]
                row16 = lane16 + g * 16
                for k in range(KPG):
                    k16 = jnp.broadcast_to(k, (16,)).astype(jnp.int32)
                    vals = plsc.load_gather(rows16, [row16, k16])
                    plsc.addupdate_scatter(acc_v, [dj16, k16], vals)

            @pl.when(t == 0)
            def _():
                for g in range(NG):
                    dj16 = didx[pl.ds(g * 16, 16)]
                    plsc.addupdate_scatter(cnt_v, [dj16], ones16)
            return carry

        lax.fori_loop(0, nchunks, chunk_body, 0)
        pltpu.sync_copy(acc_v, acc_out.at[c * NS + t])

        @pl.when(t == 0)
        def _():
            pltpu.sync_copy(cnt_v, cnt_out.at[c])

    return pl.kernel(
        body,
        out_type=[
            jax.ShapeDtypeStruct((NC * NS, nrows_pad, FG), jnp.float32),
            jax.ShapeDtypeStruct((NC, nrows_pad), jnp.float32),
        ],
        mesh=plsc.VectorSubcoreMesh(core_axis_name="c", subcore_axis_name="s"),
        scratch_types=[
            pltpu.VMEM((CHUNK,), jnp.int32),
            pltpu.VMEM((CHUNK,), jnp.int32),
            pltpu.VMEM((CHUNK,), jnp.int32),
            pltpu.VMEM((CHUNK, FG), jnp.float32),
            pltpu.VMEM((nrows_pad, FG), jnp.float32),
            pltpu.VMEM((nrows_pad,), jnp.float32),
            pltpu.SemaphoreType.DMA,
        ],
        compiler_params=pltpu.CompilerParams(needs_layout_passes=False,
                                             use_tc_tiling_on_sc=False),
    )


def _tc0_body(acc_ref, cnt_ref, xs_ref, wlT_ref, wrT_ref, b_ref, g_ref,
              be_ref, out_ref):
    acc = acc_ref[0][:N1] + acc_ref[1][:N1]
    cnt = (cnt_ref[0] + cnt_ref[1])[:N1]
    agg = acc * (1.0 / jnp.maximum(cnt, 1.0))[:, None]
    h = (jnp.dot(agg, wlT_ref[...], preferred_element_type=jnp.float32)
         + jnp.dot(xs_ref[...], wrT_ref[...], preferred_element_type=jnp.float32)
         + b_ref[...])
    mu = jnp.mean(h, axis=0, keepdims=True)
    var = jnp.mean((h - mu) ** 2, axis=0, keepdims=True)
    h = (h - mu) * lax.rsqrt(var + 1e-5) * g_ref[...] + be_ref[...]
    out_ref[...] = jnp.maximum(h, 0.0)


def _tc1_body(acc_ref, cnt_ref, h0s_ref, ci_ref, ciT_ref, wlT_ref, wrT_ref,
              b_ref, g_ref, be_ref, aT_ref, bT_ref, fcb_ref, out_ref):
    acc = acc_ref[0][:N2] + acc_ref[1][:N2]
    cnt = (cnt_ref[0] + cnt_ref[1])[:N2]
    agg = acc * (1.0 / jnp.maximum(cnt, 1.0))[:, None]
    h = (jnp.dot(agg, wlT_ref[...], preferred_element_type=jnp.float32)
         + jnp.dot(h0s_ref[...], wrT_ref[...], preferred_element_type=jnp.float32)
         + b_ref[...])
    mu = jnp.mean(h, axis=0, keepdims=True)
    var = jnp.mean((h - mu) ** 2, axis=0, keepdims=True)
    h = (h - mu) * lax.rsqrt(var + 1e-5) * g_ref[...] + be_ref[...]
    h1 = jnp.maximum(h, 0.0)

    ci = ci_ref[...]
    colsum = jnp.sum(ci, axis=0)
    cf = (jnp.dot(ciT_ref[...], h1, preferred_element_type=jnp.float32)
          * (1.0 / colsum)[:, None])
    rm = jnp.max(ci, axis=1, keepdims=True)
    io = lax.broadcasted_iota(jnp.int32, (N2, NCLUST), 1)
    amin = jnp.min(jnp.where(ci == rm, io, NCLUST), axis=1, keepdims=True)
    onehot = (io == amin).astype(jnp.float32)
    x1 = jnp.dot(onehot, cf, preferred_element_type=jnp.float32)

    aT = aT_ref[...]
    bT = bT_ref[...]
    fcb = fcb_ref[...]
    out_ref[0] = (jnp.dot(h1, aT, preferred_element_type=jnp.float32)
                  + jnp.dot(x1, bT, preferred_element_type=jnp.float32) + fcb)
    out_ref[1] = (jnp.dot(x1, aT, preferred_element_type=jnp.float32)
                  + jnp.dot(h1, bT, preferred_element_type=jnp.float32) + fcb)


def _pad_edges(ei, n_edges, nchunks, n_dst):
    per_c = n_edges // NC
    pad = nchunks * CHUNK - per_c
    src = ei[0].astype(jnp.int32).reshape(NC, per_c)
    dst = ei[1].astype(jnp.int32).reshape(NC, per_c)
    if pad:
        src = jnp.pad(src, ((0, 0), (0, pad)), constant_values=0)
        dst = jnp.pad(dst, ((0, 0), (0, pad)), constant_values=n_dst)
    return src.reshape(NC * nchunks, CHUNK), dst.reshape(NC * nchunks, CHUNK)


def kernel(x, edge_index_0, edge_index_1, cluster_index, Wl0, Wr0, b0, Wl1,
           Wr1, b1, bn_gamma0, bn_beta0, bn_gamma1, bn_beta1, fc1_W, fc1_b):
    nch0 = (E0 // NC + CHUNK - 1) // CHUNK
    nch1 = (E1 // NC + CHUNK - 1) // CHUNK
    src0, dst0 = _pad_edges(edge_index_0, E0, nch0, N1)
    src1, dst1 = _pad_edges(edge_index_1, E1, nch1, N2)

    xf = x.reshape(N0 * NS, FG)
    acc0, cnt0 = _sc_agg(N1P, nch0)(xf, src0, dst0)

    h0 = pl.pallas_call(
        _tc0_body,
        out_shape=jax.ShapeDtypeStruct((N1, HID), jnp.float32),
    )(acc0.reshape(NC, NS, N1P, FG).transpose(0, 2, 1, 3).reshape(NC, N1P, HID),
      cnt0, x[:N1], Wl0.T, Wr0.T,
      b0.reshape(1, HID), bn_gamma0.reshape(1, HID), bn_beta0.reshape(1, HID))

    acc1, cnt1 = _sc_agg(N2P, nch1)(h0.reshape(N1 * NS, FG), src1, dst1)

    out2 = pl.pallas_call(
        _tc1_body,
        out_shape=jax.ShapeDtypeStruct((2, N2, OUT_C * 4), jnp.float32),
    )(acc1.reshape(NC, NS, N2P, FG).transpose(0, 2, 1, 3).reshape(NC, N2P, HID),
      cnt1, h0[:N2], cluster_index,
      cluster_index.T, Wl1.T, Wr1.T, b1.reshape(1, HID),
      bn_gamma1.reshape(1, HID), bn_beta1.reshape(1, HID),
      fc1_W[:, :HID].T, fc1_W[:, HID:].T, fc1_b.reshape(1, OUT_C * 4))

    return out2.reshape(2 * N2, OUT_C * 4)


# speedup vs baseline: 1.6035x; 1.6035x over previous
import jax
import jax.numpy as jnp
from jax import lax
from jax.experimental import pallas as pl
from jax.experimental.pallas import tpu as pltpu
from jax.experimental.pallas import tpu_sc as plsc

IN_C = 256
HID = 256
OUT_C = 128
N0, N1, N2 = 10000, 5000, 2500
E0, E1 = 160000, 80000
NCLUST = 64

NC = 2
NS = 16
NW = NC * NS
CHUNK = 128
SUP = 512
BLKE = 4096

N1P = 5008
N2P = 2512
FG = 16


def _sc_agg(nrows_pad, nchunks):
    NGS = SUP // 16
    SPB = BLKE // SUP
    nblocks = nchunks

    def body(tablef, src_hbm, dst_hbm, acc_out, cnt_out,
             sidx_b, didx_b, fidx, rows, acc_v, cnt_v, sem0, sem1):
        c = lax.axis_index("c")
        t = lax.axis_index("s")

        zero16 = jnp.zeros((16,), jnp.float32)
        ones16 = jnp.ones((16,), jnp.float32)
        lane16 = lax.iota(jnp.int32, 16)
        t16 = jnp.broadcast_to(t, (16,)).astype(jnp.int32)

        def za(i, carry):
            for u in range(4):
                acc_v[i * 4 + u, pl.ds(0, 16)] = zero16
            return carry

        lax.fori_loop(0, nrows_pad // 4, za, 0)

        def zc(i, carry):
            cnt_v[pl.ds(i * 16, 16)] = zero16
            return carry

        lax.fori_loop(0, nrows_pad // 16, zc, 0)

        def fire(j, p):
            for g in range(NGS):
                gs = pl.ds(j * SUP + g * 16, 16)
                fidx[p, pl.ds(g * 16, 16)] = sidx_b[gs] * 16 + t16
            sem = [sem0, sem1]
            for q in range(SUP // CHUNK):
                pltpu.async_copy(
                    tablef.at[fidx.at[p, pl.ds(q * CHUNK, CHUNK)]],
                    rows.at[p, pl.ds(q * CHUNK, CHUNK)],
                    sem[p])

        def drain(p):
            sem = [sem0, sem1]
            pltpu.make_async_copy(tablef.at[pl.ds(0, SUP)], rows.at[p],
                                  sem[p]).wait()

        def block_body(b, carry):
            pltpu.sync_copy(src_hbm.at[c * nblocks + b], sidx_b)
            pltpu.sync_copy(dst_hbm.at[c * nblocks + b], didx_b)
            fire(jnp.int32(0), 0)

            def sup_body(j, carry2):
                p = j & 1

                @pl.when(j < SPB - 1)
                def _():
                    @pl.when(p == 0)
                    def _():
                        fire(j + 1, 1)

                    @pl.when(p == 1)
                    def _():
                        fire(j + 1, 0)

                @pl.when(p == 0)
                def _():
                    drain(0)

                @pl.when(p == 1)
                def _():
                    drain(1)

                for g in range(NGS):
                    dj16 = didx_b[pl.ds(j * SUP + g * 16, 16)]
                    row16 = lane16 + g * 16
                    for k in range(FG):
                        k16 = jnp.broadcast_to(k, (16,)).astype(jnp.int32)
                        vals = plsc.load_gather(rows, [row16 * 0 + p, row16,
                                                       k16])
                        plsc.addupdate_scatter(acc_v, [dj16, k16], vals)

                @pl.when(t == 0)
                def _():
                    for g in range(NGS):
                        dj16 = didx_b[pl.ds(j * SUP + g * 16, 16)]
                        plsc.addupdate_scatter(cnt_v, [dj16], ones16)
                return carry2

            lax.fori_loop(0, SPB, sup_body, 0)
            return carry

        lax.fori_loop(0, nblocks, block_body, 0)
        pltpu.sync_copy(acc_v, acc_out.at[c * NS + t])

        @pl.when(t == 0)
        def _():
            pltpu.sync_copy(cnt_v, cnt_out.at[c])

    return pl.kernel(
        body,
        out_type=[
            jax.ShapeDtypeStruct((NC * NS, nrows_pad, FG), jnp.float32),
            jax.ShapeDtypeStruct((NC, nrows_pad), jnp.float32),
        ],
        mesh=plsc.VectorSubcoreMesh(core_axis_name="c", subcore_axis_name="s"),
        scratch_types=[
            pltpu.VMEM((BLKE,), jnp.int32),
            pltpu.VMEM((BLKE,), jnp.int32),
            pltpu.VMEM((2, SUP), jnp.int32),
            pltpu.VMEM((2, SUP, FG), jnp.float32),
            pltpu.VMEM((nrows_pad, FG), jnp.float32),
            pltpu.VMEM((nrows_pad,), jnp.float32),
            pltpu.SemaphoreType.DMA,
            pltpu.SemaphoreType.DMA,
        ],
        compiler_params=pltpu.CompilerParams(needs_layout_passes=False,
                                             use_tc_tiling_on_sc=False),
    )


def _tc0_body(acc_ref, cnt_ref, xs_ref, wlT_ref, wrT_ref, b_ref, g_ref,
              be_ref, out_ref):
    acc = acc_ref[0][:N1] + acc_ref[1][:N1]
    cnt = (cnt_ref[0] + cnt_ref[1])[:N1]
    agg = acc * (1.0 / jnp.maximum(cnt, 1.0))[:, None]
    h = (jnp.dot(agg, wlT_ref[...], preferred_element_type=jnp.float32)
         + jnp.dot(xs_ref[...], wrT_ref[...], preferred_element_type=jnp.float32)
         + b_ref[...])
    mu = jnp.mean(h, axis=0, keepdims=True)
    var = jnp.mean((h - mu) ** 2, axis=0, keepdims=True)
    h = (h - mu) * lax.rsqrt(var + 1e-5) * g_ref[...] + be_ref[...]
    out_ref[...] = jnp.maximum(h, 0.0)


def _tc1_body(acc_ref, cnt_ref, h0s_ref, ci_ref, ciT_ref, wlT_ref, wrT_ref,
              b_ref, g_ref, be_ref, aT_ref, bT_ref, fcb_ref, out_ref):
    acc = acc_ref[0][:N2] + acc_ref[1][:N2]
    cnt = (cnt_ref[0] + cnt_ref[1])[:N2]
    agg = acc * (1.0 / jnp.maximum(cnt, 1.0))[:, None]
    h = (jnp.dot(agg, wlT_ref[...], preferred_element_type=jnp.float32)
         + jnp.dot(h0s_ref[...], wrT_ref[...], preferred_element_type=jnp.float32)
         + b_ref[...])
    mu = jnp.mean(h, axis=0, keepdims=True)
    var = jnp.mean((h - mu) ** 2, axis=0, keepdims=True)
    h = (h - mu) * lax.rsqrt(var + 1e-5) * g_ref[...] + be_ref[...]
    h1 = jnp.maximum(h, 0.0)

    ci = ci_ref[...]
    colsum = jnp.sum(ci, axis=0)
    cf = (jnp.dot(ciT_ref[...], h1, preferred_element_type=jnp.float32)
          * (1.0 / colsum)[:, None])
    rm = jnp.max(ci, axis=1, keepdims=True)
    io = lax.broadcasted_iota(jnp.int32, (N2, NCLUST), 1)
    amin = jnp.min(jnp.where(ci == rm, io, NCLUST), axis=1, keepdims=True)
    onehot = (io == amin).astype(jnp.float32)
    x1 = jnp.dot(onehot, cf, preferred_element_type=jnp.float32)

    aT = aT_ref[...]
    bT = bT_ref[...]
    fcb = fcb_ref[...]
    out_ref[0] = (jnp.dot(h1, aT, preferred_element_type=jnp.float32)
                  + jnp.dot(x1, bT, preferred_element_type=jnp.float32) + fcb)
    out_ref[1] = (jnp.dot(x1, aT, preferred_element_type=jnp.float32)
                  + jnp.dot(h1, bT, preferred_element_type=jnp.float32) + fcb)


def _pad_edges(ei, n_edges, nblocks, n_dst):
    per_c = n_edges // NC
    pad = nblocks * BLKE - per_c
    src = ei[0].astype(jnp.int32).reshape(NC, per_c)
    dst = ei[1].astype(jnp.int32).reshape(NC, per_c)
    if pad:
        src = jnp.pad(src, ((0, 0), (0, pad)), constant_values=0)
        dst = jnp.pad(dst, ((0, 0), (0, pad)), constant_values=n_dst)
    return src.reshape(NC * nblocks, BLKE), dst.reshape(NC * nblocks, BLKE)


def kernel(x, edge_index_0, edge_index_1, cluster_index, Wl0, Wr0, b0, Wl1,
           Wr1, b1, bn_gamma0, bn_beta0, bn_gamma1, bn_beta1, fc1_W, fc1_b):
    nch0 = (E0 // NC + BLKE - 1) // BLKE
    nch1 = (E1 // NC + BLKE - 1) // BLKE
    src0, dst0 = _pad_edges(edge_index_0, E0, nch0, N1)
    src1, dst1 = _pad_edges(edge_index_1, E1, nch1, N2)

    xf = x.reshape(N0 * NS, FG)
    acc0, cnt0 = _sc_agg(N1P, nch0)(xf, src0, dst0)

    h0 = pl.pallas_call(
        _tc0_body,
        out_shape=jax.ShapeDtypeStruct((N1, HID), jnp.float32),
    )(acc0.reshape(NC, NS, N1P, FG).transpose(0, 2, 1, 3).reshape(NC, N1P, HID),
      cnt0, x[:N1], Wl0.T, Wr0.T,
      b0.reshape(1, HID), bn_gamma0.reshape(1, HID), bn_beta0.reshape(1, HID))

    acc1, cnt1 = _sc_agg(N2P, nch1)(h0.reshape(N1 * NS, FG), src1, dst1)

    out2 = pl.pallas_call(
        _tc1_body,
        out_shape=jax.ShapeDtypeStruct((2, N2, OUT_C * 4), jnp.float32),
    )(acc1.reshape(NC, NS, N2P, FG).transpose(0, 2, 1, 3).reshape(NC, N2P, HID),
      cnt1, h0[:N2], cluster_index,
      cluster_index.T, Wl1.T, Wr1.T, b1.reshape(1, HID),
      bn_gamma1.reshape(1, HID), bn_beta1.reshape(1, HID),
      fc1_W[:, :HID].T, fc1_W[:, HID:].T, fc1_b.reshape(1, OUT_C * 4))

    return out2.reshape(2 * N2, OUT_C * 4)


# speedup vs baseline: 2.1258x; 1.3257x over previous
import jax
import jax.numpy as jnp
from jax import lax
from jax.experimental import pallas as pl
from jax.experimental.pallas import tpu as pltpu
from jax.experimental.pallas import tpu_sc as plsc

IN_C = 256
HID = 256
OUT_C = 128
N0, N1, N2 = 10000, 5000, 2500
E0, E1 = 160000, 80000
NCLUST = 64

NC = 2
NS = 16
NW = NC * NS
CHUNK = 128
SUP = 512
BLKE = 4096

N1P = 5008
N2P = 2512
FG = 16


def _sc_agg(nrows_pad, nchunks):
    NGS = SUP // 16
    SPB = BLKE // SUP
    nblocks = nchunks

    def body(tablef, src_hbm, dst_hbm, acc_out, cnt_out,
             sidx_b, didx_b, fidx, rows, acc_v, cnt_v, sem0, sem1):
        c = lax.axis_index("c")
        t = lax.axis_index("s")

        zero16 = jnp.zeros((16,), jnp.float32)
        ones16 = jnp.ones((16,), jnp.float32)
        lane16 = lax.iota(jnp.int32, 16)
        t16 = jnp.broadcast_to(t, (16,)).astype(jnp.int32)

        def za(i, carry):
            for u in range(4):
                acc_v[i * 4 + u, pl.ds(0, 16)] = zero16
            return carry

        lax.fori_loop(0, nrows_pad // 4, za, 0)

        def zc(i, carry):
            cnt_v[pl.ds(i * 16, 16)] = zero16
            return carry

        lax.fori_loop(0, nrows_pad // 16, zc, 0)

        def fire(j, p):
            for g in range(NGS):
                gs = pl.ds(j * SUP + g * 16, 16)
                fidx[p, pl.ds(g * 16, 16)] = sidx_b[gs] * 16 + t16
            sem = [sem0, sem1]
            for q in range(SUP // CHUNK):
                pltpu.async_copy(
                    tablef.at[fidx.at[p, pl.ds(q * CHUNK, CHUNK)]],
                    rows.at[p, pl.ds(q * CHUNK, CHUNK)],
                    sem[p])

        def drain(p):
            sem = [sem0, sem1]
            pltpu.make_async_copy(tablef.at[pl.ds(0, SUP)], rows.at[p],
                                  sem[p]).wait()

        def block_body(b, carry):
            pltpu.sync_copy(src_hbm.at[c * nblocks + b], sidx_b)
            pltpu.sync_copy(dst_hbm.at[c * nblocks + b], didx_b)
            fire(jnp.int32(0), 0)

            def sup_body(j, carry2):
                p = j & 1

                @pl.when(j < SPB - 1)
                def _():
                    @pl.when(p == 0)
                    def _():
                        fire(j + 1, 1)

                    @pl.when(p == 1)
                    def _():
                        fire(j + 1, 0)

                for pp in (0, 1):
                    @pl.when(p == pp)
                    def _(pp=pp):
                        drain(pp)
                        for g in range(NGS):
                            dj16 = didx_b[pl.ds(j * SUP + g * 16, 16)]
                            for e in range(16):
                                dj = dj16[e]
                                plsc.addupdate(
                                    acc_v.at[dj],
                                    rows[pp, g * 16 + e, pl.ds(0, 16)])

                @pl.when(t == 0)
                def _():
                    for g in range(NGS):
                        dj16 = didx_b[pl.ds(j * SUP + g * 16, 16)]
                        plsc.addupdate_scatter(cnt_v, [dj16], ones16)
                return carry2

            lax.fori_loop(0, SPB, sup_body, 0)
            return carry

        lax.fori_loop(0, nblocks, block_body, 0)
        pltpu.sync_copy(acc_v, acc_out.at[c * NS + t])

        @pl.when(t == 0)
        def _():
            pltpu.sync_copy(cnt_v, cnt_out.at[c])

    return pl.kernel(
        body,
        out_type=[
            jax.ShapeDtypeStruct((NC * NS, nrows_pad, FG), jnp.float32),
            jax.ShapeDtypeStruct((NC, nrows_pad), jnp.float32),
        ],
        mesh=plsc.VectorSubcoreMesh(core_axis_name="c", subcore_axis_name="s"),
        scratch_types=[
            pltpu.VMEM((BLKE,), jnp.int32),
            pltpu.VMEM((BLKE,), jnp.int32),
            pltpu.VMEM((2, SUP), jnp.int32),
            pltpu.VMEM((2, SUP, FG), jnp.float32),
            pltpu.VMEM((nrows_pad, FG), jnp.float32),
            pltpu.VMEM((nrows_pad,), jnp.float32),
            pltpu.SemaphoreType.DMA,
            pltpu.SemaphoreType.DMA,
        ],
        compiler_params=pltpu.CompilerParams(needs_layout_passes=False,
                                             use_tc_tiling_on_sc=False),
    )


def _tc0_body(acc_ref, cnt_ref, xs_ref, wlT_ref, wrT_ref, b_ref, g_ref,
              be_ref, out_ref):
    acc = acc_ref[0][:N1] + acc_ref[1][:N1]
    cnt = (cnt_ref[0] + cnt_ref[1])[:N1]
    agg = acc * (1.0 / jnp.maximum(cnt, 1.0))[:, None]
    h = (jnp.dot(agg, wlT_ref[...], preferred_element_type=jnp.float32)
         + jnp.dot(xs_ref[...], wrT_ref[...], preferred_element_type=jnp.float32)
         + b_ref[...])
    mu = jnp.mean(h, axis=0, keepdims=True)
    var = jnp.mean((h - mu) ** 2, axis=0, keepdims=True)
    h = (h - mu) * lax.rsqrt(var + 1e-5) * g_ref[...] + be_ref[...]
    out_ref[...] = jnp.maximum(h, 0.0)


def _tc1_body(acc_ref, cnt_ref, h0s_ref, ci_ref, ciT_ref, wlT_ref, wrT_ref,
              b_ref, g_ref, be_ref, aT_ref, bT_ref, fcb_ref, out_ref):
    acc = acc_ref[0][:N2] + acc_ref[1][:N2]
    cnt = (cnt_ref[0] + cnt_ref[1])[:N2]
    agg = acc * (1.0 / jnp.maximum(cnt, 1.0))[:, None]
    h = (jnp.dot(agg, wlT_ref[...], preferred_element_type=jnp.float32)
         + jnp.dot(h0s_ref[...], wrT_ref[...], preferred_element_type=jnp.float32)
         + b_ref[...])
    mu = jnp.mean(h, axis=0, keepdims=True)
    var = jnp.mean((h - mu) ** 2, axis=0, keepdims=True)
    h = (h - mu) * lax.rsqrt(var + 1e-5) * g_ref[...] + be_ref[...]
    h1 = jnp.maximum(h, 0.0)

    ci = ci_ref[...]
    colsum = jnp.sum(ci, axis=0)
    cf = (jnp.dot(ciT_ref[...], h1, preferred_element_type=jnp.float32)
          * (1.0 / colsum)[:, None])
    rm = jnp.max(ci, axis=1, keepdims=True)
    io = lax.broadcasted_iota(jnp.int32, (N2, NCLUST), 1)
    amin = jnp.min(jnp.where(ci == rm, io, NCLUST), axis=1, keepdims=True)
    onehot = (io == amin).astype(jnp.float32)
    x1 = jnp.dot(onehot, cf, preferred_element_type=jnp.float32)

    aT = aT_ref[...]
    bT = bT_ref[...]
    fcb = fcb_ref[...]
    out_ref[0] = (jnp.dot(h1, aT, preferred_element_type=jnp.float32)
                  + jnp.dot(x1, bT, preferred_element_type=jnp.float32) + fcb)
    out_ref[1] = (jnp.dot(x1, aT, preferred_element_type=jnp.float32)
                  + jnp.dot(h1, bT, preferred_element_type=jnp.float32) + fcb)


def _pad_edges(ei, n_edges, nblocks, n_dst):
    per_c = n_edges // NC
    pad = nblocks * BLKE - per_c
    src = ei[0].astype(jnp.int32).reshape(NC, per_c)
    dst = ei[1].astype(jnp.int32).reshape(NC, per_c)
    if pad:
        src = jnp.pad(src, ((0, 0), (0, pad)), constant_values=0)
        dst = jnp.pad(dst, ((0, 0), (0, pad)), constant_values=n_dst)
    return src.reshape(NC * nblocks, BLKE), dst.reshape(NC * nblocks, BLKE)


def kernel(x, edge_index_0, edge_index_1, cluster_index, Wl0, Wr0, b0, Wl1,
           Wr1, b1, bn_gamma0, bn_beta0, bn_gamma1, bn_beta1, fc1_W, fc1_b):
    nch0 = (E0 // NC + BLKE - 1) // BLKE
    nch1 = (E1 // NC + BLKE - 1) // BLKE
    src0, dst0 = _pad_edges(edge_index_0, E0, nch0, N1)
    src1, dst1 = _pad_edges(edge_index_1, E1, nch1, N2)

    xf = x.reshape(N0 * NS, FG)
    acc0, cnt0 = _sc_agg(N1P, nch0)(xf, src0, dst0)

    h0 = pl.pallas_call(
        _tc0_body,
        out_shape=jax.ShapeDtypeStruct((N1, HID), jnp.float32),
    )(acc0.reshape(NC, NS, N1P, FG).transpose(0, 2, 1, 3).reshape(NC, N1P, HID),
      cnt0, x[:N1], Wl0.T, Wr0.T,
      b0.reshape(1, HID), bn_gamma0.reshape(1, HID), bn_beta0.reshape(1, HID))

    acc1, cnt1 = _sc_agg(N2P, nch1)(h0.reshape(N1 * NS, FG), src1, dst1)

    out2 = pl.pallas_call(
        _tc1_body,
        out_shape=jax.ShapeDtypeStruct((2, N2, OUT_C * 4), jnp.float32),
    )(acc1.reshape(NC, NS, N2P, FG).transpose(0, 2, 1, 3).reshape(NC, N2P, HID),
      cnt1, h0[:N2], cluster_index,
      cluster_index.T, Wl1.T, Wr1.T, b1.reshape(1, HID),
      bn_gamma1.reshape(1, HID), bn_beta1.reshape(1, HID),
      fc1_W[:, :HID].T, fc1_W[:, HID:].T, fc1_b.reshape(1, OUT_C * 4))

    return out2.reshape(2 * N2, OUT_C * 4)


# speedup vs baseline: 2.3212x; 1.0919x over previous
import jax
import jax.numpy as jnp
from jax import lax
from jax.experimental import pallas as pl
from jax.experimental.pallas import tpu as pltpu
from jax.experimental.pallas import tpu_sc as plsc

IN_C = 256
HID = 256
OUT_C = 128
N0, N1, N2 = 10000, 5000, 2500
E0, E1 = 160000, 80000
NCLUST = 64

NC = 2
NS = 16
NW = NC * NS
CHUNK = 128
SUP = 512
BLKE = 4096

N1P = 5008
N2P = 2512
FG = 16


def _sc_agg(nrows_pad, nchunks, n_src, stage):
    NGS = SUP // 16
    SPB = BLKE // SUP
    nblocks = nchunks
    nsrc_f = n_src * NS

    def body(tablef, src_hbm, dst_hbm, acc_out, cnt_out,
             sidx_b, didx_b, fidx, rows, acc_v, cnt_v, tab_sh, sem0, sem1):
        c = lax.axis_index("c")
        t = lax.axis_index("s")

        zero16 = jnp.zeros((16,), jnp.float32)
        ones16 = jnp.ones((16,), jnp.float32)
        lane16 = lax.iota(jnp.int32, 16)
        t16 = jnp.broadcast_to(t, (16,)).astype(jnp.int32)

        def za(i, carry):
            for u in range(4):
                acc_v[i * 4 + u, pl.ds(0, 16)] = zero16
            return carry

        lax.fori_loop(0, nrows_pad // 4, za, 0)

        def zc(i, carry):
            cnt_v[pl.ds(i * 16, 16)] = zero16
            return carry

        lax.fori_loop(0, nrows_pad // 16, zc, 0)

        if stage:
            toff = t * n_src
            soff = 0
            while soff < n_src:
                piece = min(SUP, n_src - soff)
                pltpu.sync_copy(tablef.at[pl.ds(toff + soff, piece)],
                                rows.at[0, pl.ds(0, piece)])
                pltpu.sync_copy(rows.at[0, pl.ds(0, piece)],
                                tab_sh.at[pl.ds(toff + soff, piece)])
                soff += piece
            plsc.subcore_barrier()

        def fire(j, p):
            for g in range(NGS):
                gs = pl.ds(j * SUP + g * 16, 16)
                fidx[p, pl.ds(g * 16, 16)] = sidx_b[gs] * 16 + t16
            sem = [sem0, sem1]
            gsrc = tab_sh if stage else tablef
            for q in range(SUP // CHUNK):
                pltpu.async_copy(
                    gsrc.at[fidx.at[p, pl.ds(q * CHUNK, CHUNK)]],
                    rows.at[p, pl.ds(q * CHUNK, CHUNK)],
                    sem[p])

        def drain(p):
            sem = [sem0, sem1]
            pltpu.make_async_copy(tablef.at[pl.ds(0, SUP)], rows.at[p],
                                  sem[p]).wait()

        def block_body(b, carry):
            pltpu.sync_copy(src_hbm.at[c * nblocks + b], sidx_b)
            pltpu.sync_copy(dst_hbm.at[c * nblocks + b], didx_b)
            fire(jnp.int32(0), 0)

            def sup_body(j, carry2):
                p = j & 1

                @pl.when(j < SPB - 1)
                def _():
                    @pl.when(p == 0)
                    def _():
                        fire(j + 1, 1)

                    @pl.when(p == 1)
                    def _():
                        fire(j + 1, 0)

                for pp in (0, 1):
                    @pl.when(p == pp)
                    def _(pp=pp):
                        drain(pp)
                        for g in range(NGS):
                            dj16 = didx_b[pl.ds(j * SUP + g * 16, 16)]
                            for e in range(16):
                                dj = dj16[e]
                                plsc.addupdate(
                                    acc_v.at[dj],
                                    rows[pp, g * 16 + e, pl.ds(0, 16)])

                @pl.when(t == 0)
                def _():
                    for g in range(NGS):
                        dj16 = didx_b[pl.ds(j * SUP + g * 16, 16)]
                        plsc.addupdate_scatter(cnt_v, [dj16], ones16)
                return carry2

            lax.fori_loop(0, SPB, sup_body, 0)
            return carry

        lax.fori_loop(0, nblocks, block_body, 0)
        pltpu.sync_copy(acc_v, acc_out.at[c, :, pl.ds(t * FG, FG)])

        @pl.when(t == 0)
        def _():
            pltpu.sync_copy(cnt_v, cnt_out.at[c])

    return pl.kernel(
        body,
        out_type=[
            jax.ShapeDtypeStruct((NC, nrows_pad, NS * FG), jnp.float32),
            jax.ShapeDtypeStruct((NC, nrows_pad), jnp.float32),
        ],
        mesh=plsc.VectorSubcoreMesh(core_axis_name="c", subcore_axis_name="s"),
        scratch_types=[
            pltpu.VMEM((BLKE,), jnp.int32),
            pltpu.VMEM((BLKE,), jnp.int32),
            pltpu.VMEM((2, SUP), jnp.int32),
            pltpu.VMEM((2, SUP, FG), jnp.float32),
            pltpu.VMEM((nrows_pad, FG), jnp.float32),
            pltpu.VMEM((nrows_pad,), jnp.float32),
            pltpu.VMEM_SHARED(((n_src * NS) if stage else NS, FG),
                              jnp.float32),
            pltpu.SemaphoreType.DMA,
            pltpu.SemaphoreType.DMA,
        ],
        compiler_params=pltpu.CompilerParams(needs_layout_passes=False,
                                             use_tc_tiling_on_sc=False),
    )


def _tc0_body(acc_ref, cnt_ref, xs_ref, wlT_ref, wrT_ref, b_ref, g_ref,
              be_ref, out_ref):
    acc = acc_ref[0][:N1] + acc_ref[1][:N1]
    cnt = (cnt_ref[0] + cnt_ref[1])[:N1]
    agg = acc * (1.0 / jnp.maximum(cnt, 1.0))[:, None]
    h = (jnp.dot(agg, wlT_ref[...], preferred_element_type=jnp.float32)
         + jnp.dot(xs_ref[...], wrT_ref[...], preferred_element_type=jnp.float32)
         + b_ref[...])
    mu = jnp.mean(h, axis=0, keepdims=True)
    var = jnp.mean((h - mu) ** 2, axis=0, keepdims=True)
    h = (h - mu) * lax.rsqrt(var + 1e-5) * g_ref[...] + be_ref[...]
    out_ref[...] = jnp.maximum(h, 0.0)


def _tc1_body(acc_ref, cnt_ref, h0s_ref, ci_ref, ciT_ref, wlT_ref, wrT_ref,
              b_ref, g_ref, be_ref, aT_ref, bT_ref, fcb_ref, out_ref):
    acc = acc_ref[0][:N2] + acc_ref[1][:N2]
    cnt = (cnt_ref[0] + cnt_ref[1])[:N2]
    agg = acc * (1.0 / jnp.maximum(cnt, 1.0))[:, None]
    h = (jnp.dot(agg, wlT_ref[...], preferred_element_type=jnp.float32)
         + jnp.dot(h0s_ref[...], wrT_ref[...], preferred_element_type=jnp.float32)
         + b_ref[...])
    mu = jnp.mean(h, axis=0, keepdims=True)
    var = jnp.mean((h - mu) ** 2, axis=0, keepdims=True)
    h = (h - mu) * lax.rsqrt(var + 1e-5) * g_ref[...] + be_ref[...]
    h1 = jnp.maximum(h, 0.0)

    ci = ci_ref[...]
    colsum = jnp.sum(ci, axis=0)
    cf = (jnp.dot(ciT_ref[...], h1, preferred_element_type=jnp.float32)
          * (1.0 / colsum)[:, None])
    rm = jnp.max(ci, axis=1, keepdims=True)
    io = lax.broadcasted_iota(jnp.int32, (N2, NCLUST), 1)
    amin = jnp.min(jnp.where(ci == rm, io, NCLUST), axis=1, keepdims=True)
    onehot = (io == amin).astype(jnp.float32)
    x1 = jnp.dot(onehot, cf, preferred_element_type=jnp.float32)

    aT = aT_ref[...]
    bT = bT_ref[...]
    fcb = fcb_ref[...]
    out_ref[0] = (jnp.dot(h1, aT, preferred_element_type=jnp.float32)
                  + jnp.dot(x1, bT, preferred_element_type=jnp.float32) + fcb)
    out_ref[1] = (jnp.dot(x1, aT, preferred_element_type=jnp.float32)
                  + jnp.dot(h1, bT, preferred_element_type=jnp.float32) + fcb)


def _pad_edges(ei, n_edges, nblocks, n_dst):
    per_c = n_edges // NC
    pad = nblocks * BLKE - per_c
    src = ei[0].astype(jnp.int32).reshape(NC, per_c)
    dst = ei[1].astype(jnp.int32).reshape(NC, per_c)
    if pad:
        src = jnp.pad(src, ((0, 0), (0, pad)), constant_values=0)
        dst = jnp.pad(dst, ((0, 0), (0, pad)), constant_values=n_dst)
    return src.reshape(NC * nblocks, BLKE), dst.reshape(NC * nblocks, BLKE)


def kernel(x, edge_index_0, edge_index_1, cluster_index, Wl0, Wr0, b0, Wl1,
           Wr1, b1, bn_gamma0, bn_beta0, bn_gamma1, bn_beta1, fc1_W, fc1_b):
    nch0 = (E0 // NC + BLKE - 1) // BLKE
    nch1 = (E1 // NC + BLKE - 1) // BLKE
    src0, dst0 = _pad_edges(edge_index_0, E0, nch0, N1)
    src1, dst1 = _pad_edges(edge_index_1, E1, nch1, N2)

    xf = x[:N1].reshape(N1 * NS, FG)
    acc0, cnt0 = _sc_agg(N1P, nch0, N1, False)(xf, src0, dst0)

    h0 = pl.pallas_call(
        _tc0_body,
        out_shape=jax.ShapeDtypeStruct((N1, HID), jnp.float32),
    )(acc0, cnt0, x[:N1], Wl0.T, Wr0.T,
      b0.reshape(1, HID), bn_gamma0.reshape(1, HID), bn_beta0.reshape(1, HID))

    acc1, cnt1 = _sc_agg(N2P, nch1, N2, True)(h0[:N2].reshape(N2 * NS, FG),
                                              src1, dst1)

    out2 = pl.pallas_call(
        _tc1_body,
        out_shape=jax.ShapeDtypeStruct((2, N2, OUT_C * 4), jnp.float32),
    )(acc1, cnt1, h0[:N2], cluster_index,
      cluster_index.T, Wl1.T, Wr1.T, b1.reshape(1, HID),
      bn_gamma1.reshape(1, HID), bn_beta1.reshape(1, HID),
      fc1_W[:, :HID].T, fc1_W[:, HID:].T, fc1_b.reshape(1, OUT_C * 4))

    return out2.reshape(2 * N2, OUT_C * 4)
